# baseline (speedup 1.0000x reference)
import functools

import jax
import jax.numpy as jnp
from jax import lax
from jax.experimental import pallas as pl
from jax.experimental.pallas import tpu as pltpu
from jax.experimental.pallas import tpu_sc as plsc

N = 10000
E = 320000
F = 128
NG = 64
NCLS = 10
NLAYERS = 3
EPS = 1e-5

NSC = 2
NTILE = 16
NW = NSC * NTILE
EPW = E // NW
CH = 80
NCHUNK = 128
EPWP = NCHUNK * CH
EPAD = EPWP - EPW
NP = 10240
RPT = NP // NTILE


def _edge_body(x_hbm, pidx_hbm, zeros_hbm, out_hbm,
               pb0, r0, agg_sh, g0):
    c = lax.axis_index("c")
    s = lax.axis_index("s")
    w = c * NTILE + s

    pltpu.sync_copy(zeros_hbm.at[pl.ds(s * RPT, RPT)],
                    agg_sh.at[pl.ds(s * RPT, RPT)])
    plsc.subcore_barrier()

    def body(j, carry):
        pltpu.sync_copy(pidx_hbm.at[w, j], pb0)
        pltpu.async_copy(x_hbm.at[pb0.at[0]], r0, g0).wait()
        pltpu.sync_copy(r0, agg_sh.at[pb0.at[1]], add=True)
        return carry

    lax.fori_loop(0, NCHUNK, body, 0)

    plsc.subcore_barrier()
    pltpu.sync_copy(agg_sh.at[pl.ds(s * RPT, RPT)],
                    out_hbm.at[c, pl.ds(s * RPT, RPT)])


_edge_call = pl.kernel(
    _edge_body,
    out_type=jax.ShapeDtypeStruct((NSC, NP, F), jnp.float32),
    mesh=plsc.VectorSubcoreMesh(core_axis_name="c", subcore_axis_name="s"),
    scratch_types=[
        pltpu.VMEM((2, CH), jnp.int32),
        pltpu.VMEM((CH, F), jnp.float32),
        pltpu.VMEM_SHARED((NP, F), jnp.float32),
        pltpu.SemaphoreType.DMA,
    ],
)


def _embed_body(h_ref, we_ref, be_ref, o_ref):
    o_ref[...] = (jnp.dot(h_ref[...], we_ref[...],
                          preferred_element_type=jnp.float32) + be_ref[...])


_embed_call = pl.pallas_call(
    _embed_body,
    out_shape=jax.ShapeDtypeStruct((N, F), jnp.float32),
)


def _layer_body(x_ref, p_ref, w_ref, b_ref, g_ref, bt_ref, o_ref):
    agg = p_ref[0, :N, :] + p_ref[1, :N, :]
    z = 2.0 * x_ref[...] + agg
    y = jnp.dot(z, w_ref[...], preferred_element_type=jnp.float32) + b_ref[...]
    mean = jnp.mean(y, axis=0, keepdims=True)
    d = y - mean
    var = jnp.mean(d * d, axis=0, keepdims=True)
    yn = d * lax.rsqrt(var + EPS) * g_ref[...] + bt_ref[...]
    o_ref[...] = jnp.maximum(yn, 0.0)


_layer_call = pl.pallas_call(
    _layer_body,
    out_shape=jax.ShapeDtypeStruct((N, F), jnp.float32),
)


def _pool_body(x_ref, batch_ref, wfc_ref, bfc_ref, o_ref):
    gids = lax.broadcasted_iota(jnp.int32, (NG, N), 0)
    onehot = (gids == batch_ref[...]).astype(jnp.float32)
    pooled = jnp.dot(onehot, x_ref[...], preferred_element_type=jnp.float32)
    o_ref[...] = (jnp.dot(pooled, wfc_ref[...],
                          preferred_element_type=jnp.float32) + bfc_ref[...])


_pool_call = pl.pallas_call(
    _pool_body,
    out_shape=jax.ShapeDtypeStruct((NG, NCLS), jnp.float32),
)


def kernel(h, edge_index, pair_info, batch, W_emb, b_emb, W, b, gamma, beta,
           Wfc, bfc):
    srcw = pair_info[0].reshape(NW, EPW)
    dstw = pair_info[1].reshape(NW, EPW)
    pad_src = jnp.zeros((NW, EPAD), jnp.int32)
    pad_dst = jnp.broadcast_to(
        N + (jnp.arange(EPAD, dtype=jnp.int32) % (NP - N)), (NW, EPAD))
    src = jnp.concatenate([srcw, pad_src], axis=1).reshape(NW, NCHUNK, CH)
    dst = jnp.concatenate([dstw, pad_dst], axis=1).reshape(NW, NCHUNK, CH)
    pidx = jnp.stack([src, dst], axis=2)
    zeros = jnp.zeros((NP, F), jnp.float32)
    x = _embed_call(h, W_emb, b_emb.reshape(1, F))
    for l in range(NLAYERS):
        parts = _edge_call(x, pidx, zeros)
        x = _layer_call(x, parts, W[l], b[l].reshape(1, F),
                        gamma[l].reshape(1, F), beta[l].reshape(1, F))
    return _pool_call(x, batch.reshape(1, N), Wfc, bfc.reshape(1, NCLS))


# speedup vs baseline: 1.2429x; 1.2429x over previous
import functools

import jax
import jax.numpy as jnp
from jax import lax
from jax.experimental import pallas as pl
from jax.experimental.pallas import tpu as pltpu
from jax.experimental.pallas import tpu_sc as plsc

N = 10000
E = 320000
F = 128
NG = 64
NCLS = 10
NLAYERS = 3
EPS = 1e-5

NSC = 2
NTILE = 16
NW = NSC * NTILE
EPW = E // NW
CH = 80
NCHUNK = 128
EPWP = NCHUNK * CH
EPAD = EPWP - EPW
NP = 10240
RPT = NP // NTILE


def _edge_body(x_hbm, src_hbm, dst_hbm, zeros_hbm, out_hbm,
               sb0, sb1, db0, db1, r0, r1, agg_sh,
               g0, g1, i0, i1, d0, d1):
    c = lax.axis_index("c")
    s = lax.axis_index("s")
    w = c * NTILE + s
    sbuf = [sb0, sb1]
    dbuf = [db0, db1]
    rows = [r0, r1]
    gsem = [g0, g1]
    isem = [i0, i1]
    dsem = [d0, d1]

    def start_idx(b, j):
        pltpu.async_copy(src_hbm.at[w, j], sbuf[b], isem[b])
        pltpu.async_copy(dst_hbm.at[w, j], dbuf[b], dsem[b])

    def wait_src_idx(b):
        pltpu.make_async_copy(src_hbm.at[w, 0], sbuf[b], isem[b]).wait()

    def wait_dst_idx(b):
        pltpu.make_async_copy(dst_hbm.at[w, 0], dbuf[b], dsem[b]).wait()

    def start_gather(b):
        pltpu.async_copy(x_hbm.at[sbuf[b]], rows[b], gsem[b])

    def wait_gather(b):
        pltpu.make_async_copy(x_hbm.at[sbuf[b]], rows[b], gsem[b]).wait()

    def scatter(b):
        pltpu.sync_copy(rows[b], agg_sh.at[dbuf[b]], add=True)

    pltpu.sync_copy(zeros_hbm.at[pl.ds(s * RPT, RPT)],
                    agg_sh.at[pl.ds(s * RPT, RPT)])
    plsc.subcore_barrier()

    start_idx(0, 0)
    start_idx(1, 1)
    wait_src_idx(0)
    start_gather(0)

    def pair(j0, carry):
        wait_src_idx(1)
        start_gather(1)
        wait_gather(0)
        wait_dst_idx(0)
        scatter(0)
        start_idx(0, j0 + 2)
        wait_gather(1)
        wait_dst_idx(1)
        scatter(1)
        start_idx(1, j0 + 3)
        wait_src_idx(0)
        start_gather(0)
        return carry

    lax.fori_loop(0, NCHUNK // 2 - 1, lambda i, cc: pair(2 * i, cc), 0)

    wait_src_idx(1)
    start_gather(1)
    wait_gather(0)
    wait_dst_idx(0)
    scatter(0)
    wait_gather(1)
    wait_dst_idx(1)
    scatter(1)

    plsc.subcore_barrier()
    pltpu.sync_copy(agg_sh.at[pl.ds(s * RPT, RPT)],
                    out_hbm.at[c, pl.ds(s * RPT, RPT)])


_edge_call = pl.kernel(
    _edge_body,
    out_type=jax.ShapeDtypeStruct((NSC, NP, F), jnp.float32),
    mesh=plsc.VectorSubcoreMesh(core_axis_name="c", subcore_axis_name="s"),
    scratch_types=[
        pltpu.VMEM((CH,), jnp.int32),
        pltpu.VMEM((CH,), jnp.int32),
        pltpu.VMEM((CH,), jnp.int32),
        pltpu.VMEM((CH,), jnp.int32),
        pltpu.VMEM((CH, F), jnp.float32),
        pltpu.VMEM((CH, F), jnp.float32),
        pltpu.VMEM_SHARED((NP, F), jnp.float32),
        pltpu.SemaphoreType.DMA,
        pltpu.SemaphoreType.DMA,
        pltpu.SemaphoreType.DMA,
        pltpu.SemaphoreType.DMA,
        pltpu.SemaphoreType.DMA,
        pltpu.SemaphoreType.DMA,
    ],
)


def _embed_body(h_ref, we_ref, be_ref, o_ref):
    o_ref[...] = (jnp.dot(h_ref[...], we_ref[...],
                          preferred_element_type=jnp.float32) + be_ref[...])


_embed_call = pl.pallas_call(
    _embed_body,
    out_shape=jax.ShapeDtypeStruct((N, F), jnp.float32),
)


def _layer_body(x_ref, p_ref, w_ref, b_ref, g_ref, bt_ref, o_ref):
    agg = p_ref[0, :N, :] + p_ref[1, :N, :]
    z = 2.0 * x_ref[...] + agg
    y = jnp.dot(z, w_ref[...], preferred_element_type=jnp.float32) + b_ref[...]
    mean = jnp.mean(y, axis=0, keepdims=True)
    d = y - mean
    var = jnp.mean(d * d, axis=0, keepdims=True)
    yn = d * lax.rsqrt(var + EPS) * g_ref[...] + bt_ref[...]
    o_ref[...] = jnp.maximum(yn, 0.0)


_layer_call = pl.pallas_call(
    _layer_body,
    out_shape=jax.ShapeDtypeStruct((N, F), jnp.float32),
)


def _pool_body(x_ref, batch_ref, wfc_ref, bfc_ref, o_ref):
    gids = lax.broadcasted_iota(jnp.int32, (NG, N), 0)
    onehot = (gids == batch_ref[...]).astype(jnp.float32)
    pooled = jnp.dot(onehot, x_ref[...], preferred_element_type=jnp.float32)
    o_ref[...] = (jnp.dot(pooled, wfc_ref[...],
                          preferred_element_type=jnp.float32) + bfc_ref[...])


_pool_call = pl.pallas_call(
    _pool_body,
    out_shape=jax.ShapeDtypeStruct((NG, NCLS), jnp.float32),
)


def kernel(h, edge_index, pair_info, batch, W_emb, b_emb, W, b, gamma, beta,
           Wfc, bfc):
    srcw = pair_info[0].reshape(NW, EPW)
    dstw = pair_info[1].reshape(NW, EPW)
    pad_src = jnp.zeros((NW, EPAD), jnp.int32)
    pad_dst = jnp.broadcast_to(
        N + (jnp.arange(EPAD, dtype=jnp.int32) % (NP - N)), (NW, EPAD))
    src = jnp.concatenate([srcw, pad_src], axis=1).reshape(NW, NCHUNK, CH)
    dst = jnp.concatenate([dstw, pad_dst], axis=1).reshape(NW, NCHUNK, CH)
    zeros = jnp.zeros((NP, F), jnp.float32)
    x = _embed_call(h, W_emb, b_emb.reshape(1, F))
    for l in range(NLAYERS):
        parts = _edge_call(x, src, dst, zeros)
        x = _layer_call(x, parts, W[l], b[l].reshape(1, F),
                        gamma[l].reshape(1, F), beta[l].reshape(1, F))
    return _pool_call(x, batch.reshape(1, N), Wfc, bfc.reshape(1, NCLS))


# speedup vs baseline: 1.7093x; 1.3753x over previous
import functools

import jax
import jax.numpy as jnp
from jax import lax
from jax.experimental import pallas as pl
from jax.experimental.pallas import tpu as pltpu
from jax.experimental.pallas import tpu_sc as plsc

N = 10000
E = 320000
F = 128
NG = 64
NCLS = 10
NLAYERS = 3
EPS = 1e-5

NSC = 2
NTILE = 16
NW = NSC * NTILE
EPW = E // NW
CH = 80
NCHUNK = EPW // CH
NP = 10240
RPT = NP // NTILE


def _edge_body(x_hbm, src_hbm, dst_hbm, zeros_hbm, out_hbm,
               src_v, dst_v, rows_v, agg_sh, gsem):
    c = lax.axis_index("c")
    s = lax.axis_index("s")
    w = c * NTILE + s
    base = w * EPW
    pltpu.sync_copy(zeros_hbm.at[pl.ds(s * RPT, RPT)],
                    agg_sh.at[pl.ds(s * RPT, RPT)])
    plsc.subcore_barrier()

    def body(j, carry):
        off = base + j * CH
        pltpu.sync_copy(src_hbm.at[pl.ds(off, CH)], src_v)
        pltpu.sync_copy(dst_hbm.at[pl.ds(off, CH)], dst_v)
        pltpu.async_copy(x_hbm.at[src_v], rows_v, gsem).wait()
        pltpu.sync_copy(rows_v, agg_sh.at[dst_v], add=True)
        return carry

    lax.fori_loop(0, NCHUNK, body, 0)
    plsc.subcore_barrier()
    pltpu.sync_copy(agg_sh.at[pl.ds(s * RPT, RPT)],
                    out_hbm.at[c, pl.ds(s * RPT, RPT)])


_edge_call = pl.kernel(
    _edge_body,
    out_type=jax.ShapeDtypeStruct((NSC, NP, F), jnp.float32),
    mesh=plsc.VectorSubcoreMesh(core_axis_name="c", subcore_axis_name="s"),
    scratch_types=[
        pltpu.VMEM((CH,), jnp.int32),
        pltpu.VMEM((CH,), jnp.int32),
        pltpu.VMEM((CH, F), jnp.float32),
        pltpu.VMEM_SHARED((NP, F), jnp.float32),
        pltpu.SemaphoreType.DMA,
    ],
)


def _embed_body(h_ref, we_ref, be_ref, o_ref):
    o_ref[...] = (jnp.dot(h_ref[...], we_ref[...],
                          preferred_element_type=jnp.float32) + be_ref[...])


_embed_call = pl.pallas_call(
    _embed_body,
    out_shape=jax.ShapeDtypeStruct((N, F), jnp.float32),
)


def _layer_body(x_ref, p_ref, w_ref, b_ref, g_ref, bt_ref, o_ref):
    agg = p_ref[0, :N, :] + p_ref[1, :N, :]
    z = 2.0 * x_ref[...] + agg
    y = jnp.dot(z, w_ref[...], preferred_element_type=jnp.float32) + b_ref[...]
    mean = jnp.mean(y, axis=0, keepdims=True)
    d = y - mean
    var = jnp.mean(d * d, axis=0, keepdims=True)
    yn = d * lax.rsqrt(var + EPS) * g_ref[...] + bt_ref[...]
    o_ref[...] = jnp.maximum(yn, 0.0)


_layer_call = pl.pallas_call(
    _layer_body,
    out_shape=jax.ShapeDtypeStruct((N, F), jnp.float32),
)


def _pool_body(x_ref, batch_ref, wfc_ref, bfc_ref, o_ref):
    gids = lax.broadcasted_iota(jnp.int32, (NG, N), 0)
    onehot = (gids == batch_ref[...]).astype(jnp.float32)
    pooled = jnp.dot(onehot, x_ref[...], preferred_element_type=jnp.float32)
    o_ref[...] = (jnp.dot(pooled, wfc_ref[...],
                          preferred_element_type=jnp.float32) + bfc_ref[...])


_pool_call = pl.pallas_call(
    _pool_body,
    out_shape=jax.ShapeDtypeStruct((NG, NCLS), jnp.float32),
)


def kernel(h, edge_index, pair_info, batch, W_emb, b_emb, W, b, gamma, beta,
           Wfc, bfc):
    src = pair_info[0]
    dst = pair_info[1]
    zeros = jnp.zeros((NP, F), jnp.float32)
    x = _embed_call(h, W_emb, b_emb.reshape(1, F))
    for l in range(NLAYERS):
        parts = _edge_call(x, src, dst, zeros)
        x = _layer_call(x, parts, W[l], b[l].reshape(1, F),
                        gamma[l].reshape(1, F), beta[l].reshape(1, F))
    return _pool_call(x, batch.reshape(1, N), Wfc, bfc.reshape(1, NCLS))
